# Initial kernel scaffold; baseline (speedup 1.0000x reference)
import jax
import jax.numpy as jnp
from jax.experimental import pallas as pl


def kernel(x, edge_weight, W_fp, b_fp, g_fp, beta_fp, cheb_W, cheb_b, tr_W, tr_b, tr_ln_g, tr_ln_b, gcn_W, gcn_b, ep_W1, ep_b1, ep_W2, ep_b2, cp_W1, cp_b1, cp_W2, cp_b2, edge_index):
    raise NotImplementedError("write your pallas kernel here")


# speedup vs baseline: 1.8968x; 1.8968x over previous
import functools

import jax
import jax.numpy as jnp
from jax import lax
from jax.experimental import pallas as pl
from jax.experimental.pallas import tpu as pltpu
from jax.experimental.pallas import tpu_sc as plsc

N = 512
HID = 64
HEADS = 4
DH = 16
N_LAYERS = 3
T_STEPS = 10
E = 16384

NC = 2
NS = 16
NW = NC * NS
EPT = E // NW
CHUNK = 128
NCHUNK = EPT // CHUNK
ZCH = (N * N) // NS

_f32 = jnp.float32


def _sc_count_body(src_hbm, dst_hbm, zeros_hbm, cnt_out,
                   src_v, dst_v, fi2, ones2, cacc):
  c = lax.axis_index("c")
  s = lax.axis_index("s")
  wid = s * NC + c
  base = wid * EPT

  off = s * ZCH
  pltpu.sync_copy(zeros_hbm.at[pl.ds(0, ZCH)], cacc.at[pl.ds(off, ZCH)])

  pltpu.sync_copy(src_hbm.at[pl.ds(base, EPT)], src_v)
  pltpu.sync_copy(dst_hbm.at[pl.ds(base, EPT)], dst_v)

  for k in range(NCHUNK):
    for j in range(CHUNK // 16):
      sl = pl.ds(k * CHUNK + j * 16, 16)
      sj = pl.ds(j * 16, 16)
      fi2[k, sj] = dst_v[sl] * N + src_v[sl]
      ones2[k, sj] = jnp.full((16,), 1.0, _f32)

  plsc.subcore_barrier()

  for k in range(NCHUNK):
    pltpu.sync_copy(ones2.at[k], cacc.at[fi2.at[k]], add=True)

  plsc.subcore_barrier()

  pltpu.sync_copy(cacc.at[pl.ds(off, ZCH)], cnt_out.at[c, pl.ds(off, ZCH)])


@functools.lru_cache(maxsize=1)
def _sc_count():
  return functools.partial(
      pl.kernel,
      out_type=jax.ShapeDtypeStruct((NC, N * N), _f32),
      mesh=plsc.VectorSubcoreMesh(core_axis_name="c", subcore_axis_name="s"),
      scratch_types=[
          pltpu.VMEM((EPT,), jnp.int32),
          pltpu.VMEM((EPT,), jnp.int32),
          pltpu.VMEM((NCHUNK, CHUNK), jnp.int32),
          pltpu.VMEM((NCHUNK, CHUNK), _f32),
          pltpu.VMEM_SHARED((N * N,), _f32),
      ],
  )(_sc_count_body)


_DIMS = (((1,), (0,)), ((), ()))
_DIMS_T = (((1,), (1,)), ((), ()))


def _mm(a, b):
  return lax.dot_general(a, b, _DIMS, precision=lax.Precision.HIGHEST,
                         preferred_element_type=_f32)


def _mmt(a, b):
  return lax.dot_general(a, b, _DIMS_T, precision=lax.Precision.HIGHEST,
                         preferred_element_type=_f32)


def _mm_lo(a, b):
  return lax.dot_general(a.astype(jnp.bfloat16), b.astype(jnp.bfloat16),
                         _DIMS, preferred_element_type=_f32)


def _mmt_lo(a, b):
  return lax.dot_general(a.astype(jnp.bfloat16), b.astype(jnp.bfloat16),
                         _DIMS_T, preferred_element_type=_f32)


def _gelu(v):
  return v * (lax.erf(v / 1.4142135623730951) + 1.0) / 2.0


def _tc_tail_body(h_ref, cnt_ref, trWh_ref, trbh_ref, trWs_ref, trbs_ref,
                  lng_ref, lnb_ref, gcnW_ref, gcnb_ref,
                  epA_ref, epB_ref, epb1_ref, w2k_ref, epb2_ref,
                  cpW1_ref, cpb1_ref, cpw2_ref, cpb2_ref,
                  adj_ref, cp_ref, agg_sc):
  h = h_ref[...]
  Cm = cnt_ref[0] + cnt_ref[1]
  mask = Cm > 0.0
  neg = jnp.float32(-1e30)

  degg = jnp.sum(Cm, axis=1) + 1.0
  dgi = 1.0 / jnp.sqrt(degg)
  rows = lax.broadcasted_iota(jnp.int32, (N, N), 0)
  cols = lax.broadcasted_iota(jnp.int32, (N, N), 1)
  Gm = Cm * (dgi[:, None] * dgi[None, :]) + jnp.where(
      rows == cols, dgi[:, None] * dgi[:, None], 0.0)

  def head_body(hd, carry):
    qkv = _mm_lo(h, trWh_ref[hd]) + trbh_ref[hd]
    qh = qkv[:, :DH]
    kh = qkv[:, DH:2 * DH]
    vh = qkv[:, 2 * DH:]
    S = _mmt(qh, kh) * 0.25
    amax = jnp.max(jnp.where(mask, S, neg), axis=1)
    arg = jnp.where(mask, S - amax[:, None], -60.0)
    Wm = Cm * jnp.exp(arg)
    asum = jnp.sum(Wm, axis=1)
    asum = jnp.where(asum == 0.0, 1.0, asum)
    agg_sc[hd] = _mm(Wm / asum[:, None], vh)
    return carry

  lax.fori_loop(0, HEADS, head_body, 0)
  aggr = jnp.concatenate([agg_sc[0], agg_sc[1], agg_sc[2], agg_sc[3]], axis=1)
  h2 = aggr + (_mm_lo(h, trWs_ref[...]) + trbs_ref[...])
  mu = jnp.mean(h2, axis=1, keepdims=True)
  var = jnp.mean((h2 - mu) * (h2 - mu), axis=1, keepdims=True)
  h2 = (h2 - mu) / jnp.sqrt(var + 1e-5) * lng_ref[...] + lnb_ref[...]
  xw = _mm_lo(h2, gcnW_ref[...])
  h2 = _mm(Gm, xw) + gcnb_ref[...]

  A = _mm_lo(h2, epA_ref[...]) + epb1_ref[...]
  Bm = _mm_lo(h2, epB_ref[...])
  Bt = jnp.concatenate([Bm] * 8, axis=1)
  w2k = w2k_ref[...]
  epb2 = epb2_ref[0, 0]
  lane_iota = lax.broadcasted_iota(jnp.int32, (1, N), 1)

  def row_step(r, carry):
    rows8 = [(lane_iota == 8 * r + q).astype(_f32) for q in range(8)]
    arow = jnp.concatenate([_mm(oh, A) for oh in rows8], axis=1)
    g = _gelu(Bt + arow)
    out8 = _mmt_lo(w2k, g) + epb2
    adj_ref[0, pl.ds(pl.multiple_of(r * 8, 8), 8), :] = out8
    return carry

  lax.fori_loop(0, N // 8, row_step, 0)

  cph = _gelu(_mm_lo(h2, cpW1_ref[...]) + cpb1_ref[...])
  cpv = _mm_lo(cph, cpw2_ref[...]) + cpb2_ref[0, 0]
  cp_ref[...] = cpv.reshape(1, N)


def _tc_tail(h, cnt_p, trWh, trbh, trWs, trbs, lng, lnb, gcnW, gcnb,
             epA, epB, epb1, w2k, epb2, cpW1, cpb1, cpw2, cpb2):
  return pl.pallas_call(
      _tc_tail_body,
      out_shape=[
          jax.ShapeDtypeStruct((1, N, N), _f32),
          jax.ShapeDtypeStruct((1, N), _f32),
      ],
      scratch_shapes=[
          pltpu.VMEM((HEADS, N, DH), _f32),
      ],
  )(h, cnt_p, trWh, trbh, trWs, trbs, lng, lnb, gcnW, gcnb,
    epA, epB, epb1, w2k, epb2, cpW1, cpb1, cpw2, cpb2)


def kernel(x, edge_weight, W_fp, b_fp, g_fp, beta_fp, cheb_W, cheb_b,
           tr_W, tr_b, tr_ln_g, tr_ln_b, gcn_W, gcn_b,
           ep_W1, ep_b1, ep_W2, ep_b2, cp_W1, cp_b1, cp_W2, cp_b2,
           edge_index):
  src = edge_index[0]
  dst = edge_index[1]
  B = x.shape[0]

  zeros = jnp.zeros((ZCH,), _f32)
  cnt_p = _sc_count()(src, dst, zeros).reshape(NC, N, N)

  mask = (src != dst).astype(jnp.float32)
  w_eff = edge_weight * mask
  deg = jnp.zeros((N,), jnp.float32).at[src].add(w_eff)
  dis = jnp.where(deg > 0, 1.0 / jnp.sqrt(jnp.where(deg > 0, deg, 1.0)), 0.0)
  lw = -(dis[src] * w_eff * dis[dst])

  def prop(v):
    return jnp.zeros_like(v).at[:, dst, :].add(lw[None, :, None] * v[:, src, :])

  def cheb(v, W, b):
    out = v @ W[0]
    t1 = prop(v)
    out = out + t1 @ W[1]
    t2 = 2.0 * prop(t1) - v
    out = out + t2 @ W[2]
    return out + b

  h = jnp.zeros((B, N, HID), jnp.float32)
  for _ in range(T_STEPS):
    Z = jax.nn.sigmoid(cheb(h, cheb_W[0], cheb_b[0]) + cheb_b[1])
    Ht = jnp.tanh(cheb(h, cheb_W[4], cheb_b[4]) + cheb_b[5])
    h = (1.0 - Z) * Ht

  loop = jnp.arange(N, dtype=src.dtype)
  src2 = jnp.concatenate([src, loop])
  dst2 = jnp.concatenate([dst, loop])
  degg = jnp.zeros((N,), jnp.float32).at[dst2].add(
      jnp.ones((src2.shape[0],), jnp.float32))
  dgi = 1.0 / jnp.sqrt(degg)
  gnorm = dgi[src2] * dgi[dst2]
  for l in range(N_LAYERS - 1):
    q = (h @ tr_W[l, 0] + tr_b[l, 0]).reshape(B, N, HEADS, DH)
    k = (h @ tr_W[l, 1] + tr_b[l, 1]).reshape(B, N, HEADS, DH)
    v = (h @ tr_W[l, 2] + tr_b[l, 2]).reshape(B, N, HEADS, DH)
    alpha = (q[:, dst] * k[:, src]).sum(-1) / jnp.sqrt(float(DH))
    amax = jnp.full((B, N, HEADS), -1e30, jnp.float32).at[:, dst].max(alpha)
    aexp = jnp.exp(alpha - amax[:, dst])
    asum = jnp.zeros((B, N, HEADS), jnp.float32).at[:, dst].add(aexp)
    asum = jnp.where(asum == 0.0, 1.0, asum)
    att = aexp / asum[:, dst]
    aggr = jnp.zeros((B, N, HEADS, DH), jnp.float32).at[:, dst].add(
        att[..., None] * v[:, src]).reshape(B, N, HID)
    h = aggr + (h @ tr_W[l, 3] + tr_b[l, 3])
    mu = h.mean(-1, keepdims=True)
    var = ((h - mu) ** 2).mean(-1, keepdims=True)
    h = (h - mu) / jnp.sqrt(var + 1e-5) * tr_ln_g[l] + tr_ln_b[l]
    xw = h @ gcn_W[l]
    h = jnp.zeros((B, N, HID), jnp.float32).at[:, dst2, :].add(
        gnorm[None, :, None] * xw[:, src2, :]) + gcn_b[l]

  lf = N_LAYERS - 1
  trWh = (tr_W[lf, :3].reshape(3, HID, HEADS, DH)
          .transpose(2, 1, 0, 3).reshape(HEADS, HID, 3 * DH))
  trbh = (tr_b[lf, :3].reshape(3, HEADS, DH)
          .transpose(1, 0, 2).reshape(HEADS, 1, 3 * DH))
  w2k = jnp.kron(jnp.eye(8, dtype=_f32), ep_W2.reshape(1, HID))

  adj, cp = _tc_tail(
      h[0], cnt_p, trWh, trbh, tr_W[lf, 3], tr_b[lf, 3].reshape(1, HID),
      tr_ln_g[lf].reshape(1, HID), tr_ln_b[lf].reshape(1, HID),
      gcn_W[lf], gcn_b[lf].reshape(1, HID),
      ep_W1[:HID], ep_W1[HID:], ep_b1.reshape(1, HID), w2k,
      ep_b2.reshape(1, 1), cp_W1, cp_b1.reshape(1, HID), cp_W2,
      cp_b2.reshape(1, 1))
  return adj, cp
